# baseline (speedup 1.0000x reference)
import jax
import jax.numpy as jnp
from jax import lax
from jax.experimental import pallas as pl
from jax.experimental.pallas import tpu as pltpu
from jax.experimental.pallas import tpu_sc as plsc

N = 10000
E = 320000
D = 128
H = 8
DH = 16

NC = 2
NS = 16
NW = NC * NS
EPW = E // NW
C = 40
NCHUNK = EPW // C
NP = 10112
RPT = NP // NS

NB = 2000
GRID = N // NB


def _sel_matrix(rows, cols, group):
    r = lax.broadcasted_iota(jnp.int32, (rows, cols), 0)
    c = lax.broadcasted_iota(jnp.int32, (rows, cols), 1)
    return (c // group == r).astype(jnp.float32)


def _pair_perm_matrix():
    r = lax.broadcasted_iota(jnp.int32, (D, D), 0)
    p = lax.broadcasted_iota(jnp.int32, (D, D), 1)
    src = 32 * (p // 32) + 16 * (p % 2) + (p % 32) // 2
    return (r == src).astype(jnp.float32)


def _prologue_body(h_ref, w_ref, al_ref, ar_ref, ei_ref,
                   featb_ref, elp_ref, erp_ref, src_ref, dst_ref):
    @pl.when(pl.program_id(0) == 0)
    def _():
        src_ref[...] = ei_ref[0, :]
        dst_ref[...] = ei_ref[1, :]
    feat = jnp.dot(h_ref[...], w_ref[...], preferred_element_type=jnp.float32)
    sel_t = _sel_matrix(H, D, DH).T
    el = jnp.dot(feat * al_ref[...], sel_t, preferred_element_type=jnp.float32)
    er = jnp.dot(feat * ar_ref[...], sel_t, preferred_element_type=jnp.float32)
    zs = jnp.zeros((feat.shape[0], 8), jnp.float32)
    featp = jnp.dot(feat, _pair_perm_matrix(),
                    preferred_element_type=jnp.float32)
    featb_ref[...] = featp.astype(jnp.bfloat16)
    elp_ref[...] = jnp.concatenate([el, zs], axis=1)
    erp_ref[...] = jnp.concatenate([er, zs], axis=1)


def _epilogue_body(num_ref, den_ref, h_ref, bias_ref, out_ref):
    num = num_ref[0] + num_ref[1]
    d = den_ref[0] + den_ref[1]
    den = jnp.maximum(d[:, 0:H], 1e-9)
    denf = jnp.dot(1.0 / den, _sel_matrix(H, D, DH),
                   preferred_element_type=jnp.float32)
    x = num * denf + bias_ref[...]
    o = jnp.where(x > 0.0, x, jnp.exp(jnp.minimum(x, 0.0)) - 1.0)
    out_ref[...] = h_ref[...] + o


def _sc_body(featb_hbm, elp_hbm, erp_hbm, srcr_hbm, dstr_hbm, zn_hbm, zd_hbm,
             outn_hbm, outd_hbm,
             sidx, didx, fbb0, fbb1, eb0, eb1, fb0, fb1, rb0, rb1, rb2,
             accn, accd,
             gf0, gf1, ge0, ge1, gr0, gr1, gr2, sn0, sn1, sd0, sd1, sd2):
    fbbs = (fbb0, fbb1)
    ebs = (eb0, eb1)
    fbs = (fb0, fb1)
    rbs = (rb0, rb1, rb2)
    gfs = (gf0, gf1)
    ges = (ge0, ge1)
    grs = (gr0, gr1, gr2)
    sns = (sn0, sn1)
    sds = (sd0, sd1, sd2)
    c = lax.axis_index("c")
    s = lax.axis_index("s")
    wid = c * NS + s

    cz = pltpu.async_copy(zn_hbm, accn.at[pl.ds(s * RPT, RPT)], gf0)
    cw = pltpu.async_copy(zd_hbm, accd.at[pl.ds(s * RPT, RPT)], ge0)
    ci = pltpu.async_copy(srcr_hbm.at[wid], sidx, gr0)
    cj = pltpu.async_copy(dstr_hbm.at[wid], didx, sn0)
    cz.wait()
    cw.wait()
    ci.wait()
    cj.wait()
    plsc.subcore_barrier()

    def start_gather(i, a, r):
        pltpu.async_copy(featb_hbm.at[sidx.at[pl.ds(i * C, C)]], fbbs[a],
                         gfs[a])
        pltpu.async_copy(elp_hbm.at[sidx.at[pl.ds(i * C, C)]], ebs[a], ges[a])
        pltpu.async_copy(erp_hbm.at[didx.at[i]], rbs[r], grs[r])

    def wait_gather(a, r):
        pltpu.make_async_copy(featb_hbm.at[sidx.at[pl.ds(0, C)]], fbbs[a],
                              gfs[a]).wait()
        pltpu.make_async_copy(elp_hbm.at[sidx.at[pl.ds(0, C)]], ebs[a],
                              ges[a]).wait()
        pltpu.make_async_copy(erp_hbm.at[didx.at[0]], rbs[r], grs[r]).wait()

    def compute(a, r):
        fbb = fbbs[a]
        eb = ebs[a]
        fb = fbs[a]
        rb = rbs[r]

        @plsc.parallel_loop(0, C, unroll=8)
        def _edge(e):
            t = eb[e, :] + rb[e, :]
            t = jnp.maximum(t, t * 0.2)
            w = jnp.exp(t)
            rb[e, :] = w
            for g in range(4):
                x = fbb[e, pl.ds(g * 32, 32)]
                va, vb = plsc.unpack(x, format=plsc.PackFormat.INTERLEAVED)
                fb[e, pl.ds(g * 32, DH)] = va * w[2 * g]
                fb[e, pl.ds(g * 32 + DH, DH)] = vb * w[2 * g + 1]

    def sub_step(i, a, r, first, last):
        a2 = 1 - a
        r2 = (r + 1) % 3
        if not first:
            pltpu.make_async_copy(rbs[r2], accd.at[didx.at[0]],
                                  sds[r2]).wait()
        if not last:
            start_gather(i + 1, a2, r2)
        wait_gather(a, r)
        if not first:
            pltpu.make_async_copy(fbs[a], accn.at[didx.at[0]], sns[a]).wait()
        compute(a, r)
        pltpu.async_copy(fbs[a], accn.at[didx.at[i]], sns[a], add=True)
        pltpu.async_copy(rbs[r], accd.at[didx.at[i]], sds[r], add=True)

    start_gather(0, 0, 0)
    sub_step(0, 0, 0, True, False)
    sub_step(1, 1, 1, True, False)

    @pl.loop(0, (NCHUNK - 4) // 6)
    def _pipe(j):
        i = 6 * j + 2
        sub_step(i + 0, 0, 2, False, False)
        sub_step(i + 1, 1, 0, False, False)
        sub_step(i + 2, 0, 1, False, False)
        sub_step(i + 3, 1, 2, False, False)
        sub_step(i + 4, 0, 0, False, False)
        sub_step(i + 5, 1, 1, False, False)

    sub_step(NCHUNK - 2, 0, 2, False, False)
    sub_step(NCHUNK - 1, 1, 0, False, True)
    pltpu.make_async_copy(fbs[0], accn.at[didx.at[0]], sns[0]).wait()
    pltpu.make_async_copy(fbs[1], accn.at[didx.at[0]], sns[1]).wait()
    pltpu.make_async_copy(rbs[2], accd.at[didx.at[0]], sds[2]).wait()
    pltpu.make_async_copy(rbs[0], accd.at[didx.at[0]], sds[0]).wait()

    plsc.subcore_barrier()
    pltpu.sync_copy(accn.at[pl.ds(s * RPT, RPT)],
                    outn_hbm.at[c, pl.ds(s * RPT, RPT)])
    pltpu.sync_copy(accd.at[pl.ds(s * RPT, RPT)],
                    outd_hbm.at[c, pl.ds(s * RPT, RPT)])


def kernel(h, edge_index, W, attn_l, attn_r, bias):
    al = attn_l.reshape(1, D)
    ar = attn_r.reshape(1, D)
    b = bias.reshape(1, D)

    EB = E // GRID
    featb, elp, erp, src1, dst1 = pl.pallas_call(
        _prologue_body,
        grid=(GRID,),
        in_specs=[
            pl.BlockSpec((NB, D), lambda i: (i, 0)),
            pl.BlockSpec((D, D), lambda i: (0, 0)),
            pl.BlockSpec((1, D), lambda i: (0, 0)),
            pl.BlockSpec((1, D), lambda i: (0, 0)),
            pl.BlockSpec((2, E), lambda i: (0, 0)),
        ],
        out_specs=[
            pl.BlockSpec((NB, D), lambda i: (i, 0)),
            pl.BlockSpec((NB, DH), lambda i: (i, 0)),
            pl.BlockSpec((NB, DH), lambda i: (i, 0)),
            pl.BlockSpec((E,), lambda i: (0,)),
            pl.BlockSpec((E,), lambda i: (0,)),
        ],
        out_shape=[
            jax.ShapeDtypeStruct((N, D), jnp.bfloat16),
            jax.ShapeDtypeStruct((N, DH), jnp.float32),
            jax.ShapeDtypeStruct((N, DH), jnp.float32),
            jax.ShapeDtypeStruct((E,), jnp.int32),
            jax.ShapeDtypeStruct((E,), jnp.int32),
        ],
    )(h, W, al, ar, edge_index)

    srcr = src1.reshape(NW, EPW)
    dstr = dst1.reshape(NW, NCHUNK, C)
    zn = jnp.zeros((RPT, D), jnp.float32)
    zd = jnp.zeros((RPT, DH), jnp.float32)

    mesh = plsc.VectorSubcoreMesh(core_axis_name="c", subcore_axis_name="s")
    sc_fn = pl.kernel(
        _sc_body,
        out_type=[
            jax.ShapeDtypeStruct((NC, NP, D), jnp.float32),
            jax.ShapeDtypeStruct((NC, NP, DH), jnp.float32),
        ],
        mesh=mesh,
        scratch_types=(
            [pltpu.VMEM((EPW,), jnp.int32)]
            + [pltpu.VMEM((NCHUNK, C), jnp.int32)]
            + [pltpu.VMEM((C, D), jnp.bfloat16)] * 2
            + [pltpu.VMEM((C, DH), jnp.float32)] * 2
            + [pltpu.VMEM((C, D), jnp.float32)] * 2
            + [pltpu.VMEM((C, DH), jnp.float32)] * 3
            + [pltpu.VMEM_SHARED((NP, D), jnp.float32)]
            + [pltpu.VMEM_SHARED((NP, DH), jnp.float32)]
            + [pltpu.SemaphoreType.DMA] * 12
        ),
        compiler_params=pltpu.CompilerParams(use_tc_tiling_on_sc=False,
                                             needs_layout_passes=False),
    )
    outn, outd = sc_fn(featb, elp, erp, srcr, dstr, zn, zd)

    out = pl.pallas_call(
        _epilogue_body,
        grid=(GRID,),
        in_specs=[
            pl.BlockSpec((NC, NB, D), lambda i: (0, i, 0)),
            pl.BlockSpec((NC, NB, DH), lambda i: (0, i, 0)),
            pl.BlockSpec((NB, D), lambda i: (i, 0)),
            pl.BlockSpec((1, D), lambda i: (0, 0)),
        ],
        out_specs=pl.BlockSpec((NB, D), lambda i: (i, 0)),
        out_shape=jax.ShapeDtypeStruct((N, D), jnp.float32),
    )(outn, outd, h, b)
    return out


# speedup vs baseline: 1.1450x; 1.1450x over previous
import jax
import jax.numpy as jnp
from jax import lax
from jax.experimental import pallas as pl
from jax.experimental.pallas import tpu as pltpu
from jax.experimental.pallas import tpu_sc as plsc

N = 10000
E = 320000
D = 128
H = 8
DH = 16

NC = 2
NS = 16
NW = NC * NS
EPW = E // NW
C = 40
NCHUNK = EPW // C
NP = 10112
RPT = NP // NS

NB = 2000
GRID = N // NB


def _sel_matrix(rows, cols, group):
    r = lax.broadcasted_iota(jnp.int32, (rows, cols), 0)
    c = lax.broadcasted_iota(jnp.int32, (rows, cols), 1)
    return (c // group == r).astype(jnp.float32)


def _pair_perm_matrix():
    r = lax.broadcasted_iota(jnp.int32, (D, D), 0)
    p = lax.broadcasted_iota(jnp.int32, (D, D), 1)
    src = 32 * (p // 32) + 16 * (p % 2) + (p % 32) // 2
    return (r == src).astype(jnp.float32)


def _prologue_body(h_ref, w_ref, al_ref, ar_ref, ei_ref,
                   featb_ref, elp_ref, erp_ref, src_ref, dst_ref):
    @pl.when(pl.program_id(0) == 0)
    def _():
        src_ref[...] = ei_ref[0, :]
        dst_ref[...] = ei_ref[1, :]
    feat = jnp.dot(h_ref[...], w_ref[...], preferred_element_type=jnp.float32)
    sel_t = _sel_matrix(H, D, DH).T
    el = jnp.dot(feat * al_ref[...], sel_t, preferred_element_type=jnp.float32)
    er = jnp.dot(feat * ar_ref[...], sel_t, preferred_element_type=jnp.float32)
    zs = jnp.zeros((feat.shape[0], 8), jnp.float32)
    featp = jnp.dot(feat, _pair_perm_matrix(),
                    preferred_element_type=jnp.float32)
    featb_ref[...] = featp.astype(jnp.bfloat16)
    elp_ref[...] = jnp.concatenate([el, zs], axis=1)
    erp_ref[...] = jnp.concatenate([er, zs], axis=1)


def _epilogue_body(num_ref, den_ref, h_ref, bias_ref, out_ref):
    num = num_ref[0] + num_ref[1]
    d = den_ref[0] + den_ref[1]
    den = jnp.maximum(d[:, 0:H], 1e-9)
    denf = jnp.dot(1.0 / den, _sel_matrix(H, D, DH),
                   preferred_element_type=jnp.float32)
    x = num * denf + bias_ref[...]
    o = jnp.where(x > 0.0, x, jnp.exp(jnp.minimum(x, 0.0)) - 1.0)
    out_ref[...] = h_ref[...] + o


def _sc_body(featb_hbm, elp_hbm, erp_hbm, srcr_hbm, dstr_hbm, zn_hbm, zd_hbm,
             outn_hbm, outd_hbm,
             sidx, didx, fbb0, fbb1, eb0, eb1, fb0, fb1, rb0, rb1, rb2,
             accn, accd,
             gf0, gf1, ge0, ge1, gr0, gr1, gr2, sn0, sn1, sd0, sd1, sd2):
    fbbs = (fbb0, fbb1)
    ebs = (eb0, eb1)
    fbs = (fb0, fb1)
    rbs = (rb0, rb1, rb2)
    gfs = (gf0, gf1)
    ges = (ge0, ge1)
    grs = (gr0, gr1, gr2)
    sns = (sn0, sn1)
    sds = (sd0, sd1, sd2)
    c = lax.axis_index("c")
    s = lax.axis_index("s")
    wid = c * NS + s

    cz = pltpu.async_copy(zn_hbm, accn.at[pl.ds(s * RPT, RPT)], gf0)
    cw = pltpu.async_copy(zd_hbm, accd.at[pl.ds(s * RPT, RPT)], ge0)
    ci = pltpu.async_copy(srcr_hbm.at[wid], sidx, gr0)
    cj = pltpu.async_copy(dstr_hbm.at[wid], didx, sn0)
    cz.wait()
    cw.wait()
    ci.wait()
    cj.wait()
    plsc.subcore_barrier()

    def start_gather(i, a, r):
        pltpu.async_copy(featb_hbm.at[sidx.at[pl.ds(i * C, C)]], fbbs[a],
                         gfs[a])
        pltpu.async_copy(elp_hbm.at[sidx.at[pl.ds(i * C, C)]], ebs[a], ges[a])
        pltpu.async_copy(erp_hbm.at[didx.at[i]], rbs[r], grs[r])

    def wait_gather(a, r):
        pltpu.make_async_copy(featb_hbm.at[sidx.at[pl.ds(0, C)]], fbbs[a],
                              gfs[a]).wait()
        pltpu.make_async_copy(elp_hbm.at[sidx.at[pl.ds(0, C)]], ebs[a],
                              ges[a]).wait()
        pltpu.make_async_copy(erp_hbm.at[didx.at[0]], rbs[r], grs[r]).wait()

    def compute(a, r):
        fbb = fbbs[a]
        eb = ebs[a]
        fb = fbs[a]
        rb = rbs[r]

        @plsc.parallel_loop(0, C, unroll=2)
        def _edge(e):
            t = eb[e, :] + rb[e, :]
            t = jnp.maximum(t, t * 0.2)
            w = jnp.exp(t)
            rb[e, :] = w
            for g in range(4):
                x = fbb[e, pl.ds(g * 32, 32)]
                va, vb = plsc.unpack(x, format=plsc.PackFormat.INTERLEAVED)
                fb[e, pl.ds(g * 32, DH)] = va * w[2 * g]
                fb[e, pl.ds(g * 32 + DH, DH)] = vb * w[2 * g + 1]

    def sub_step(i, a, r, first, last):
        a2 = 1 - a
        r2 = (r + 1) % 3
        if not first:
            pltpu.make_async_copy(rbs[r2], accd.at[didx.at[0]],
                                  sds[r2]).wait()
        if not last:
            start_gather(i + 1, a2, r2)
        wait_gather(a, r)
        if not first:
            pltpu.make_async_copy(fbs[a], accn.at[didx.at[0]], sns[a]).wait()
        compute(a, r)
        pltpu.async_copy(fbs[a], accn.at[didx.at[i]], sns[a], add=True)
        pltpu.async_copy(rbs[r], accd.at[didx.at[i]], sds[r], add=True)

    start_gather(0, 0, 0)
    sub_step(0, 0, 0, True, False)
    sub_step(1, 1, 1, True, False)

    @pl.loop(0, (NCHUNK - 4) // 6)
    def _pipe(j):
        i = 6 * j + 2
        sub_step(i + 0, 0, 2, False, False)
        sub_step(i + 1, 1, 0, False, False)
        sub_step(i + 2, 0, 1, False, False)
        sub_step(i + 3, 1, 2, False, False)
        sub_step(i + 4, 0, 0, False, False)
        sub_step(i + 5, 1, 1, False, False)

    sub_step(NCHUNK - 2, 0, 2, False, False)
    sub_step(NCHUNK - 1, 1, 0, False, True)
    pltpu.make_async_copy(fbs[0], accn.at[didx.at[0]], sns[0]).wait()
    pltpu.make_async_copy(fbs[1], accn.at[didx.at[0]], sns[1]).wait()
    pltpu.make_async_copy(rbs[2], accd.at[didx.at[0]], sds[2]).wait()
    pltpu.make_async_copy(rbs[0], accd.at[didx.at[0]], sds[0]).wait()

    plsc.subcore_barrier()
    pltpu.sync_copy(accn.at[pl.ds(s * RPT, RPT)],
                    outn_hbm.at[c, pl.ds(s * RPT, RPT)])
    pltpu.sync_copy(accd.at[pl.ds(s * RPT, RPT)],
                    outd_hbm.at[c, pl.ds(s * RPT, RPT)])


def kernel(h, edge_index, W, attn_l, attn_r, bias):
    al = attn_l.reshape(1, D)
    ar = attn_r.reshape(1, D)
    b = bias.reshape(1, D)

    EB = E // GRID
    featb, elp, erp, src1, dst1 = pl.pallas_call(
        _prologue_body,
        grid=(GRID,),
        in_specs=[
            pl.BlockSpec((NB, D), lambda i: (i, 0)),
            pl.BlockSpec((D, D), lambda i: (0, 0)),
            pl.BlockSpec((1, D), lambda i: (0, 0)),
            pl.BlockSpec((1, D), lambda i: (0, 0)),
            pl.BlockSpec((2, E), lambda i: (0, 0)),
        ],
        out_specs=[
            pl.BlockSpec((NB, D), lambda i: (i, 0)),
            pl.BlockSpec((NB, DH), lambda i: (i, 0)),
            pl.BlockSpec((NB, DH), lambda i: (i, 0)),
            pl.BlockSpec((E,), lambda i: (0,)),
            pl.BlockSpec((E,), lambda i: (0,)),
        ],
        out_shape=[
            jax.ShapeDtypeStruct((N, D), jnp.bfloat16),
            jax.ShapeDtypeStruct((N, DH), jnp.float32),
            jax.ShapeDtypeStruct((N, DH), jnp.float32),
            jax.ShapeDtypeStruct((E,), jnp.int32),
            jax.ShapeDtypeStruct((E,), jnp.int32),
        ],
    )(h, W, al, ar, edge_index)

    srcr = src1.reshape(NW, EPW)
    dstr = dst1.reshape(NW, NCHUNK, C)
    zn = jnp.zeros((RPT, D), jnp.float32)
    zd = jnp.zeros((RPT, DH), jnp.float32)

    mesh = plsc.VectorSubcoreMesh(core_axis_name="c", subcore_axis_name="s")
    sc_fn = pl.kernel(
        _sc_body,
        out_type=[
            jax.ShapeDtypeStruct((NC, NP, D), jnp.float32),
            jax.ShapeDtypeStruct((NC, NP, DH), jnp.float32),
        ],
        mesh=mesh,
        scratch_types=(
            [pltpu.VMEM((EPW,), jnp.int32)]
            + [pltpu.VMEM((NCHUNK, C), jnp.int32)]
            + [pltpu.VMEM((C, D), jnp.bfloat16)] * 2
            + [pltpu.VMEM((C, DH), jnp.float32)] * 2
            + [pltpu.VMEM((C, D), jnp.float32)] * 2
            + [pltpu.VMEM((C, DH), jnp.float32)] * 3
            + [pltpu.VMEM_SHARED((NP, D), jnp.float32)]
            + [pltpu.VMEM_SHARED((NP, DH), jnp.float32)]
            + [pltpu.SemaphoreType.DMA] * 12
        ),
        compiler_params=pltpu.CompilerParams(use_tc_tiling_on_sc=False,
                                             needs_layout_passes=False),
    )
    outn, outd = sc_fn(featb, elp, erp, srcr, dstr, zn, zd)

    out = pl.pallas_call(
        _epilogue_body,
        grid=(GRID,),
        in_specs=[
            pl.BlockSpec((NC, NB, D), lambda i: (0, i, 0)),
            pl.BlockSpec((NC, NB, DH), lambda i: (0, i, 0)),
            pl.BlockSpec((NB, D), lambda i: (i, 0)),
            pl.BlockSpec((1, D), lambda i: (0, 0)),
        ],
        out_specs=pl.BlockSpec((NB, D), lambda i: (i, 0)),
        out_shape=jax.ShapeDtypeStruct((N, D), jnp.float32),
    )(outn, outd, h, b)
    return out
